# baseline (device time: 373949 ns/iter reference)
import os

import jax
import jax.numpy as jnp
from jax import lax
from jax.experimental import pallas as pl
from jax.experimental.pallas import tpu as pltpu

N_DEV = 32
B, SQ, HQ_LOC, DH = 2, 512, 8, 64
D_MODEL = 768
ROWS = B * SQ
G8 = 8
NP = N_DEV // G8
C8 = ROWS // G8
C4 = C8 // NP
N_SEM = (G8 - 1) + 2 * (NP - 1) + (G8 - 1)
_SKIP_COMM = os.environ.get("SKIP_COMM") == "1"


def kernel(x, Wq, K_ext, V_ext, Wo):
    x2 = x.reshape(ROWS, D_MODEL)
    k2 = K_ext.reshape(B, SQ, 256 * DH)
    v2 = V_ext.reshape(B, SQ, 256 * DH)

    def body(x_ref, wq_ref, k_ref, v_ref, wo_ref, out_ref,
             kv_ref, vv_ref, comm8_ref, comm4_ref,
             kv_sems, vv_sems, send_sems, recv_sems):
        my = lax.axis_index("i")
        plane = my // G8
        q = lax.rem(my, G8)
        pl_right = plane * G8 + lax.rem(q + 1, G8)
        pl_left = plane * G8 + lax.rem(q + G8 - 1, G8)
        z_right = lax.rem(plane + 1, NP) * G8 + q
        z_left = lax.rem(plane + NP - 1, NP) * G8 + q

        barrier_sem = pltpu.get_barrier_semaphore()
        for nbr in (pl_left, pl_right, z_left, z_right):
            pl.semaphore_signal(barrier_sem, inc=1, device_id=(nbr,),
                                device_id_type=pl.DeviceIdType.MESH)
        pl.semaphore_wait(barrier_sem, 4)

        col0 = my * (HQ_LOC * DH)
        kv_copies = []
        for src, dst, sems in ((k_ref, kv_ref, kv_sems),
                               (v_ref, vv_ref, vv_sems)):
            cp = pltpu.make_async_copy(
                src.at[:, :, pl.ds(col0, HQ_LOC * DH)],
                dst.at[:, :, :],
                sems.at[0],
            )
            cp.start()
            kv_copies.append(cp)

        qmat = jnp.dot(x_ref[:, :], wq_ref[:, :],
                       preferred_element_type=jnp.float32)
        qb = lax.broadcasted_iota(jnp.int32, (SQ, SQ), 0) // 64
        kb = lax.broadcasted_iota(jnp.int32, (SQ, SQ), 1) // 64
        mask = (qb == kb) | (kb == 0) | ((qb + kb) % 3 == 0)
        for cp in kv_copies:
            cp.wait()
        for b in range(B):
            qrows = qmat[b * SQ:(b + 1) * SQ, :]
            heads = []
            for h in range(HQ_LOC):
                qh = qrows[:, h * DH:(h + 1) * DH]
                kh = kv_ref[b, :, h * DH:(h + 1) * DH]
                vh = vv_ref[b, :, h * DH:(h + 1) * DH]
                s = lax.dot_general(qh, kh, (((1,), (1,)), ((), ())),
                                    preferred_element_type=jnp.float32)
                s = jnp.where(mask, s * 0.125, jnp.float32(-1e9))
                m = jnp.max(s, axis=1, keepdims=True)
                w = jnp.exp(s - m)
                w = w / jnp.sum(w, axis=1, keepdims=True)
                heads.append(jnp.dot(w, vh,
                                     preferred_element_type=jnp.float32))
            ctx = jnp.concatenate(heads, axis=1)
            out_ref[b * SQ:(b + 1) * SQ, :] = jnp.dot(
                ctx, wo_ref[:, :], preferred_element_type=jnp.float32)

        for st in range(0 if _SKIP_COMM else G8 - 1):
            send_c = lax.rem(q + 2 * G8 - st, G8)
            recv_c = lax.rem(q + 2 * G8 - st - 1, G8)
            rdma = pltpu.make_async_remote_copy(
                src_ref=out_ref.at[pl.ds(send_c * C8, C8), :],
                dst_ref=comm8_ref.at[st],
                send_sem=send_sems.at[st],
                recv_sem=recv_sems.at[st],
                device_id=(pl_right,),
                device_id_type=pl.DeviceIdType.MESH,
            )
            rdma.start()
            rdma.wait()
            out_ref[pl.ds(recv_c * C8, C8), :] = (
                out_ref[pl.ds(recv_c * C8, C8), :] + comm8_ref[st])

        o8 = lax.rem(q + 1, G8)
        base = o8 * C8
        for st in range(0 if _SKIP_COMM else NP - 1):
            send_s = lax.rem(plane + 2 * NP - st, NP)
            recv_s = lax.rem(plane + 2 * NP - st - 1, NP)
            rdma = pltpu.make_async_remote_copy(
                src_ref=out_ref.at[pl.ds(base + send_s * C4, C4), :],
                dst_ref=comm4_ref.at[st],
                send_sem=send_sems.at[7 + st],
                recv_sem=recv_sems.at[7 + st],
                device_id=(z_right,),
                device_id_type=pl.DeviceIdType.MESH,
            )
            rdma.start()
            rdma.wait()
            out_ref[pl.ds(base + recv_s * C4, C4), :] = (
                out_ref[pl.ds(base + recv_s * C4, C4), :] + comm4_ref[st])
        for st in range(0 if _SKIP_COMM else NP - 1):
            send_s = lax.rem(plane + 1 + 2 * NP - st, NP)
            off = base + send_s * C4
            rdma = pltpu.make_async_remote_copy(
                src_ref=out_ref.at[pl.ds(off, C4), :],
                dst_ref=out_ref.at[pl.ds(off, C4), :],
                send_sem=send_sems.at[10 + st],
                recv_sem=recv_sems.at[10 + st],
                device_id=(z_right,),
                device_id_type=pl.DeviceIdType.MESH,
            )
            rdma.start()
            rdma.wait()

        for st in range(0 if _SKIP_COMM else G8 - 1):
            send_c = lax.rem(q + 1 + 2 * G8 - st, G8)
            off = send_c * C8
            rdma = pltpu.make_async_remote_copy(
                src_ref=out_ref.at[pl.ds(off, C8), :],
                dst_ref=out_ref.at[pl.ds(off, C8), :],
                send_sem=send_sems.at[13 + st],
                recv_sem=recv_sems.at[13 + st],
                device_id=(pl_right,),
                device_id_type=pl.DeviceIdType.MESH,
            )
            rdma.start()
            rdma.wait()

    out2 = pl.pallas_call(
        body,
        out_shape=jax.ShapeDtypeStruct((ROWS, D_MODEL), jnp.float32),
        in_specs=[
            pl.BlockSpec(memory_space=pltpu.VMEM),
            pl.BlockSpec(memory_space=pltpu.VMEM),
            pl.BlockSpec(memory_space=pl.ANY),
            pl.BlockSpec(memory_space=pl.ANY),
            pl.BlockSpec(memory_space=pltpu.VMEM),
        ],
        out_specs=pl.BlockSpec(memory_space=pltpu.VMEM),
        scratch_shapes=[
            pltpu.VMEM((B, SQ, HQ_LOC * DH), jnp.float32),
            pltpu.VMEM((B, SQ, HQ_LOC * DH), jnp.float32),
            pltpu.VMEM((G8 - 1, C8, D_MODEL), jnp.float32),
            pltpu.VMEM((NP - 1, C4, D_MODEL), jnp.float32),
            pltpu.SemaphoreType.DMA((1,)),
            pltpu.SemaphoreType.DMA((1,)),
            pltpu.SemaphoreType.DMA((N_SEM,)),
            pltpu.SemaphoreType.DMA((N_SEM,)),
        ],
        compiler_params=pltpu.CompilerParams(collective_id=0),
    )(x2, Wq, k2, v2, Wo)
    return out2.reshape(B, SQ, D_MODEL)


# device time: 258528 ns/iter; 1.4465x vs baseline; 1.4465x over previous
import os

import jax
import jax.numpy as jnp
from jax import lax
from jax.experimental import pallas as pl
from jax.experimental.pallas import tpu as pltpu

N_DEV = 32
B, SQ, HQ_LOC, DH = 2, 512, 8, 64
D_MODEL = 768
ROWS = B * SQ
G8 = 8
NP = N_DEV // G8
C8 = ROWS // G8
C4 = C8 // NP
N_SEM = (G8 - 1) + 2 * (NP - 1) + (G8 - 1)
_SKIP_COMM = os.environ.get("SKIP_COMM") == "1"


def kernel(x, Wq, K_ext, V_ext, Wo):
    x2 = x.reshape(ROWS, D_MODEL)
    HD = HQ_LOC * DH
    idx = lax.axis_index("i")
    k_loc = lax.dynamic_slice(
        K_ext.reshape(B, SQ, N_DEV, HD), (0, 0, idx, 0),
        (B, SQ, 1, HD)).reshape(B, SQ, HD)
    v_loc = lax.dynamic_slice(
        V_ext.reshape(B, SQ, N_DEV, HD), (0, 0, idx, 0),
        (B, SQ, 1, HD)).reshape(B, SQ, HD)

    def body(x_ref, wq_ref, k_ref, v_ref, wo_ref, out_ref,
             comm8_ref, comm4_ref, send_sems, recv_sems):
        my = lax.axis_index("i")
        plane = my // G8
        q = lax.rem(my, G8)
        pl_right = plane * G8 + lax.rem(q + 1, G8)
        pl_left = plane * G8 + lax.rem(q + G8 - 1, G8)
        z_right = lax.rem(plane + 1, NP) * G8 + q
        z_left = lax.rem(plane + NP - 1, NP) * G8 + q

        barrier_sem = pltpu.get_barrier_semaphore()
        for nbr in (pl_left, pl_right, z_left, z_right):
            pl.semaphore_signal(barrier_sem, inc=1, device_id=(nbr,),
                                device_id_type=pl.DeviceIdType.MESH)
        pl.semaphore_wait(barrier_sem, 4)

        qmat = jnp.dot(x_ref[:, :], wq_ref[:, :],
                       preferred_element_type=jnp.float32)
        qb = lax.broadcasted_iota(jnp.int32, (SQ, SQ), 0) // 64
        kb = lax.broadcasted_iota(jnp.int32, (SQ, SQ), 1) // 64
        mask = (qb == kb) | (kb == 0) | ((qb + kb) % 3 == 0)
        for b in range(B):
            qrows = qmat[b * SQ:(b + 1) * SQ, :]
            kall = k_ref[b, :, :]
            vall = v_ref[b, :, :]
            heads = []
            for h in range(HQ_LOC):
                qh = qrows[:, h * DH:(h + 1) * DH]
                kh = kall[:, h * DH:(h + 1) * DH]
                vh = vall[:, h * DH:(h + 1) * DH]
                s = lax.dot_general(qh, kh, (((1,), (1,)), ((), ())),
                                    preferred_element_type=jnp.float32)
                s = jnp.where(mask, s * 0.125, jnp.float32(-1e9))
                m = jnp.max(s, axis=1, keepdims=True)
                w = jnp.exp(s - m)
                w = w / jnp.sum(w, axis=1, keepdims=True)
                heads.append(jnp.dot(w, vh,
                                     preferred_element_type=jnp.float32))
            ctx = jnp.concatenate(heads, axis=1)
            out_ref[b * SQ:(b + 1) * SQ, :] = jnp.dot(
                ctx, wo_ref[:, :], preferred_element_type=jnp.float32)

        for st in range(0 if _SKIP_COMM else G8 - 1):
            send_c = lax.rem(q + 2 * G8 - st, G8)
            recv_c = lax.rem(q + 2 * G8 - st - 1, G8)
            rdma = pltpu.make_async_remote_copy(
                src_ref=out_ref.at[pl.ds(send_c * C8, C8), :],
                dst_ref=comm8_ref.at[st],
                send_sem=send_sems.at[st],
                recv_sem=recv_sems.at[st],
                device_id=(pl_right,),
                device_id_type=pl.DeviceIdType.MESH,
            )
            rdma.start()
            rdma.wait()
            out_ref[pl.ds(recv_c * C8, C8), :] = (
                out_ref[pl.ds(recv_c * C8, C8), :] + comm8_ref[st])

        o8 = lax.rem(q + 1, G8)
        base = o8 * C8
        for st in range(0 if _SKIP_COMM else NP - 1):
            send_s = lax.rem(plane + 2 * NP - st, NP)
            recv_s = lax.rem(plane + 2 * NP - st - 1, NP)
            rdma = pltpu.make_async_remote_copy(
                src_ref=out_ref.at[pl.ds(base + send_s * C4, C4), :],
                dst_ref=comm4_ref.at[st],
                send_sem=send_sems.at[7 + st],
                recv_sem=recv_sems.at[7 + st],
                device_id=(z_right,),
                device_id_type=pl.DeviceIdType.MESH,
            )
            rdma.start()
            rdma.wait()
            out_ref[pl.ds(base + recv_s * C4, C4), :] = (
                out_ref[pl.ds(base + recv_s * C4, C4), :] + comm4_ref[st])
        for st in range(0 if _SKIP_COMM else NP - 1):
            send_s = lax.rem(plane + 1 + 2 * NP - st, NP)
            off = base + send_s * C4
            rdma = pltpu.make_async_remote_copy(
                src_ref=out_ref.at[pl.ds(off, C4), :],
                dst_ref=out_ref.at[pl.ds(off, C4), :],
                send_sem=send_sems.at[10 + st],
                recv_sem=recv_sems.at[10 + st],
                device_id=(z_right,),
                device_id_type=pl.DeviceIdType.MESH,
            )
            rdma.start()
            rdma.wait()

        for st in range(0 if _SKIP_COMM else G8 - 1):
            send_c = lax.rem(q + 1 + 2 * G8 - st, G8)
            off = send_c * C8
            rdma = pltpu.make_async_remote_copy(
                src_ref=out_ref.at[pl.ds(off, C8), :],
                dst_ref=out_ref.at[pl.ds(off, C8), :],
                send_sem=send_sems.at[13 + st],
                recv_sem=recv_sems.at[13 + st],
                device_id=(pl_right,),
                device_id_type=pl.DeviceIdType.MESH,
            )
            rdma.start()
            rdma.wait()

    out2 = pl.pallas_call(
        body,
        out_shape=jax.ShapeDtypeStruct((ROWS, D_MODEL), jnp.float32),
        in_specs=[pl.BlockSpec(memory_space=pltpu.VMEM)] * 5,
        out_specs=pl.BlockSpec(memory_space=pltpu.VMEM),
        scratch_shapes=[
            pltpu.VMEM((G8 - 1, C8, D_MODEL), jnp.float32),
            pltpu.VMEM((NP - 1, C4, D_MODEL), jnp.float32),
            pltpu.SemaphoreType.DMA((N_SEM,)),
            pltpu.SemaphoreType.DMA((N_SEM,)),
        ],
        compiler_params=pltpu.CompilerParams(collective_id=0),
    )(x2, Wq, k_loc, v_loc, Wo)
    return out2.reshape(B, SQ, D_MODEL)


# device time: 253081 ns/iter; 1.4776x vs baseline; 1.0215x over previous
import os

import jax
import jax.numpy as jnp
from jax import lax
from jax.experimental import pallas as pl
from jax.experimental.pallas import tpu as pltpu

N_DEV = 32
B, SQ, HQ_LOC, DH = 2, 512, 8, 64
D_MODEL = 768
ROWS = B * SQ
G8 = 8
NP = N_DEV // G8
C8 = ROWS // G8
C4 = C8 // NP
N_SEM = (G8 - 1) + 2 * (NP - 1) + (G8 - 1)
_SKIP_COMM = os.environ.get("SKIP_COMM") == "1"


def kernel(x, Wq, K_ext, V_ext, Wo):
    x2 = x.reshape(ROWS, D_MODEL)
    HD = HQ_LOC * DH
    idx = lax.axis_index("i")
    k_loc = lax.dynamic_slice(
        K_ext.reshape(B, SQ, N_DEV, HD), (0, 0, idx, 0),
        (B, SQ, 1, HD)).reshape(B, SQ, HD)
    v_loc = lax.dynamic_slice(
        V_ext.reshape(B, SQ, N_DEV, HD), (0, 0, idx, 0),
        (B, SQ, 1, HD)).reshape(B, SQ, HD)

    def body(x_ref, wq_ref, k_ref, v_ref, wo_ref, out_ref,
             comm8_ref, comm4_ref, send_sems, recv_sems):
        my = lax.axis_index("i")
        plane = my // G8
        q = lax.rem(my, G8)
        pl_right = plane * G8 + lax.rem(q + 1, G8)
        pl_left = plane * G8 + lax.rem(q + G8 - 1, G8)
        z_right = lax.rem(plane + 1, NP) * G8 + q
        z_left = lax.rem(plane + NP - 1, NP) * G8 + q

        barrier_sem = pltpu.get_barrier_semaphore()
        for nbr in (pl_left, pl_right, z_left, z_right):
            pl.semaphore_signal(barrier_sem, inc=1, device_id=(nbr,),
                                device_id_type=pl.DeviceIdType.MESH)
        pl.semaphore_wait(barrier_sem, 4)

        qmat = jnp.dot(x_ref[:, :], wq_ref[:, :],
                       preferred_element_type=jnp.float32)
        qb = lax.broadcasted_iota(jnp.int32, (SQ, SQ), 0) // 64
        kb = lax.broadcasted_iota(jnp.int32, (SQ, SQ), 1) // 64
        mask = (qb == kb) | (kb == 0) | ((qb + kb) % 3 == 0)
        cm0 = (lax.broadcasted_iota(jnp.int32, (1, 2 * DH), 1)
               < DH).astype(jnp.float32)
        cm1 = 1.0 - cm0
        for b in range(B):
            qrows = qmat[b * SQ:(b + 1) * SQ, :]
            kall = k_ref[b, :, :]
            vall = v_ref[b, :, :]
            pairs = []
            for hp in range(HQ_LOC // 2):
                sl = slice(hp * 2 * DH, (hp + 1) * 2 * DH)
                qp = qrows[:, sl]
                kp = kall[:, sl]
                vp = vall[:, sl]
                ctx_p = None
                for cm in (cm0, cm1):
                    s = lax.dot_general(qp * cm, kp,
                                        (((1,), (1,)), ((), ())),
                                        preferred_element_type=jnp.float32)
                    s = jnp.where(mask, s * 0.125, jnp.float32(-1e9))
                    m = jnp.max(s, axis=1, keepdims=True)
                    w = jnp.exp(s - m)
                    w = w / jnp.sum(w, axis=1, keepdims=True)
                    c = jnp.dot(w, vp * cm,
                                preferred_element_type=jnp.float32)
                    ctx_p = c if ctx_p is None else ctx_p + c
                pairs.append(ctx_p)
            ctx = jnp.concatenate(pairs, axis=1)
            out_ref[b * SQ:(b + 1) * SQ, :] = jnp.dot(
                ctx, wo_ref[:, :], preferred_element_type=jnp.float32)

        for st in range(0 if _SKIP_COMM else G8 - 1):
            send_c = lax.rem(q + 2 * G8 - st, G8)
            recv_c = lax.rem(q + 2 * G8 - st - 1, G8)
            rdma = pltpu.make_async_remote_copy(
                src_ref=out_ref.at[pl.ds(send_c * C8, C8), :],
                dst_ref=comm8_ref.at[st],
                send_sem=send_sems.at[st],
                recv_sem=recv_sems.at[st],
                device_id=(pl_right,),
                device_id_type=pl.DeviceIdType.MESH,
            )
            rdma.start()
            rdma.wait()
            out_ref[pl.ds(recv_c * C8, C8), :] = (
                out_ref[pl.ds(recv_c * C8, C8), :] + comm8_ref[st])

        o8 = lax.rem(q + 1, G8)
        base = o8 * C8
        for st in range(0 if _SKIP_COMM else NP - 1):
            send_s = lax.rem(plane + 2 * NP - st, NP)
            recv_s = lax.rem(plane + 2 * NP - st - 1, NP)
            rdma = pltpu.make_async_remote_copy(
                src_ref=out_ref.at[pl.ds(base + send_s * C4, C4), :],
                dst_ref=comm4_ref.at[st],
                send_sem=send_sems.at[7 + st],
                recv_sem=recv_sems.at[7 + st],
                device_id=(z_right,),
                device_id_type=pl.DeviceIdType.MESH,
            )
            rdma.start()
            rdma.wait()
            out_ref[pl.ds(base + recv_s * C4, C4), :] = (
                out_ref[pl.ds(base + recv_s * C4, C4), :] + comm4_ref[st])
        for st in range(0 if _SKIP_COMM else NP - 1):
            send_s = lax.rem(plane + 1 + 2 * NP - st, NP)
            off = base + send_s * C4
            rdma = pltpu.make_async_remote_copy(
                src_ref=out_ref.at[pl.ds(off, C4), :],
                dst_ref=out_ref.at[pl.ds(off, C4), :],
                send_sem=send_sems.at[10 + st],
                recv_sem=recv_sems.at[10 + st],
                device_id=(z_right,),
                device_id_type=pl.DeviceIdType.MESH,
            )
            rdma.start()
            rdma.wait()

        for st in range(0 if _SKIP_COMM else G8 - 1):
            send_c = lax.rem(q + 1 + 2 * G8 - st, G8)
            off = send_c * C8
            rdma = pltpu.make_async_remote_copy(
                src_ref=out_ref.at[pl.ds(off, C8), :],
                dst_ref=out_ref.at[pl.ds(off, C8), :],
                send_sem=send_sems.at[13 + st],
                recv_sem=recv_sems.at[13 + st],
                device_id=(pl_right,),
                device_id_type=pl.DeviceIdType.MESH,
            )
            rdma.start()
            rdma.wait()

    out2 = pl.pallas_call(
        body,
        out_shape=jax.ShapeDtypeStruct((ROWS, D_MODEL), jnp.float32),
        in_specs=[pl.BlockSpec(memory_space=pltpu.VMEM)] * 5,
        out_specs=pl.BlockSpec(memory_space=pltpu.VMEM),
        scratch_shapes=[
            pltpu.VMEM((G8 - 1, C8, D_MODEL), jnp.float32),
            pltpu.VMEM((NP - 1, C4, D_MODEL), jnp.float32),
            pltpu.SemaphoreType.DMA((N_SEM,)),
            pltpu.SemaphoreType.DMA((N_SEM,)),
        ],
        compiler_params=pltpu.CompilerParams(collective_id=0),
    )(x2, Wq, k_loc, v_loc, Wo)
    return out2.reshape(B, SQ, D_MODEL)


# device time: 223495 ns/iter; 1.6732x vs baseline; 1.1324x over previous
import os

import jax
import jax.numpy as jnp
from jax import lax
from jax.experimental import pallas as pl
from jax.experimental.pallas import tpu as pltpu

N_DEV = 32
B, SQ, HQ_LOC, DH = 2, 512, 8, 64
D_MODEL = 768
ROWS = B * SQ
G8 = 8
NP = N_DEV // G8
C8 = ROWS // G8
C4 = C8 // NP
N_SEM = (G8 - 1) + 2 * (NP - 1) + (G8 - 1)
_SKIP_COMM = os.environ.get("SKIP_COMM") == "1"
_N_PAIRS = int(os.environ.get("N_PAIRS", HQ_LOC // 2))


def kernel(x, Wq, K_ext, V_ext, Wo):
    x2 = x.reshape(ROWS, D_MODEL)
    HD = HQ_LOC * DH
    idx = lax.axis_index("i")
    k4 = lax.dynamic_slice_in_dim(K_ext, idx * HQ_LOC, HQ_LOC, axis=2)
    v4 = lax.dynamic_slice_in_dim(V_ext, idx * HQ_LOC, HQ_LOC, axis=2)
    k4, v4 = lax.optimization_barrier((k4, v4))
    k_loc = k4.reshape(B, SQ, HD)
    v_loc = v4.reshape(B, SQ, HD)

    def body(x_ref, wq_ref, k_ref, v_ref, wo_ref, out_ref,
             comm8_ref, comm4_ref, send_sems, recv_sems):
        my = lax.axis_index("i")
        plane = my // G8
        q = lax.rem(my, G8)
        pl_right = plane * G8 + lax.rem(q + 1, G8)
        pl_left = plane * G8 + lax.rem(q + G8 - 1, G8)
        z_right = lax.rem(plane + 1, NP) * G8 + q
        z_left = lax.rem(plane + NP - 1, NP) * G8 + q

        barrier_sem = pltpu.get_barrier_semaphore()
        for nbr in (pl_left, pl_right, z_left, z_right):
            pl.semaphore_signal(barrier_sem, inc=1, device_id=(nbr,),
                                device_id_type=pl.DeviceIdType.MESH)
        pl.semaphore_wait(barrier_sem, 4)

        qmat = jnp.dot(x_ref[:, :], wq_ref[:, :],
                       preferred_element_type=jnp.float32)
        qb = lax.broadcasted_iota(jnp.int32, (SQ, SQ), 0) // 64
        kb = lax.broadcasted_iota(jnp.int32, (SQ, SQ), 1) // 64
        mask = (qb == kb) | (kb == 0) | ((qb + kb) % 3 == 0)
        cm0 = (lax.broadcasted_iota(jnp.int32, (1, 2 * DH), 1)
               < DH).astype(jnp.float32)
        cm1 = 1.0 - cm0
        for b in range(B):
            qrows = qmat[b * SQ:(b + 1) * SQ, :]
            pairs = []
            for hp in range(_N_PAIRS):
                sl = slice(hp * 2 * DH, (hp + 1) * 2 * DH)
                qp = qrows[:, sl]
                kp = k_ref[b, :, sl]
                vp = v_ref[b, :, sl]
                ctx_p = None
                for cm in (cm0, cm1):
                    s = lax.dot_general(qp * cm, kp,
                                        (((1,), (1,)), ((), ())),
                                        preferred_element_type=jnp.float32)
                    s = jnp.where(mask, s * 0.125, jnp.float32(-1e9))
                    m = jnp.max(s, axis=1, keepdims=True)
                    w = jnp.exp(s - m)
                    w = w / jnp.sum(w, axis=1, keepdims=True)
                    c = jnp.dot(w, vp * cm,
                                preferred_element_type=jnp.float32)
                    ctx_p = c if ctx_p is None else ctx_p + c
                pairs.append(ctx_p)
            if _N_PAIRS < HQ_LOC // 2:
                pairs = pairs * ((HQ_LOC // 2) // _N_PAIRS)
            ctx = jnp.concatenate(pairs, axis=1)
            out_ref[b * SQ:(b + 1) * SQ, :] = jnp.dot(
                ctx, wo_ref[:, :], preferred_element_type=jnp.float32)

        for st in range(0 if _SKIP_COMM else G8 - 1):
            send_c = lax.rem(q + 2 * G8 - st, G8)
            recv_c = lax.rem(q + 2 * G8 - st - 1, G8)
            rdma = pltpu.make_async_remote_copy(
                src_ref=out_ref.at[pl.ds(send_c * C8, C8), :],
                dst_ref=comm8_ref.at[st],
                send_sem=send_sems.at[st],
                recv_sem=recv_sems.at[st],
                device_id=(pl_right,),
                device_id_type=pl.DeviceIdType.MESH,
            )
            rdma.start()
            rdma.wait()
            out_ref[pl.ds(recv_c * C8, C8), :] = (
                out_ref[pl.ds(recv_c * C8, C8), :] + comm8_ref[st])

        o8 = lax.rem(q + 1, G8)
        base = o8 * C8
        for st in range(0 if _SKIP_COMM else NP - 1):
            send_s = lax.rem(plane + 2 * NP - st, NP)
            recv_s = lax.rem(plane + 2 * NP - st - 1, NP)
            rdma = pltpu.make_async_remote_copy(
                src_ref=out_ref.at[pl.ds(base + send_s * C4, C4), :],
                dst_ref=comm4_ref.at[st],
                send_sem=send_sems.at[7 + st],
                recv_sem=recv_sems.at[7 + st],
                device_id=(z_right,),
                device_id_type=pl.DeviceIdType.MESH,
            )
            rdma.start()
            rdma.wait()
            out_ref[pl.ds(base + recv_s * C4, C4), :] = (
                out_ref[pl.ds(base + recv_s * C4, C4), :] + comm4_ref[st])
        for st in range(0 if _SKIP_COMM else NP - 1):
            send_s = lax.rem(plane + 1 + 2 * NP - st, NP)
            off = base + send_s * C4
            rdma = pltpu.make_async_remote_copy(
                src_ref=out_ref.at[pl.ds(off, C4), :],
                dst_ref=out_ref.at[pl.ds(off, C4), :],
                send_sem=send_sems.at[10 + st],
                recv_sem=recv_sems.at[10 + st],
                device_id=(z_right,),
                device_id_type=pl.DeviceIdType.MESH,
            )
            rdma.start()
            rdma.wait()

        for st in range(0 if _SKIP_COMM else G8 - 1):
            send_c = lax.rem(q + 1 + 2 * G8 - st, G8)
            off = send_c * C8
            rdma = pltpu.make_async_remote_copy(
                src_ref=out_ref.at[pl.ds(off, C8), :],
                dst_ref=out_ref.at[pl.ds(off, C8), :],
                send_sem=send_sems.at[13 + st],
                recv_sem=recv_sems.at[13 + st],
                device_id=(pl_right,),
                device_id_type=pl.DeviceIdType.MESH,
            )
            rdma.start()
            rdma.wait()

    out2 = pl.pallas_call(
        body,
        out_shape=jax.ShapeDtypeStruct((ROWS, D_MODEL), jnp.float32),
        in_specs=[pl.BlockSpec(memory_space=pltpu.VMEM)] * 5,
        out_specs=pl.BlockSpec(memory_space=pltpu.VMEM),
        scratch_shapes=[
            pltpu.VMEM((G8 - 1, C8, D_MODEL), jnp.float32),
            pltpu.VMEM((NP - 1, C4, D_MODEL), jnp.float32),
            pltpu.SemaphoreType.DMA((N_SEM,)),
            pltpu.SemaphoreType.DMA((N_SEM,)),
        ],
        compiler_params=pltpu.CompilerParams(collective_id=0),
    )(x2, Wq, k_loc, v_loc, Wo)
    return out2.reshape(B, SQ, D_MODEL)
